# Initial kernel scaffold; baseline (speedup 1.0000x reference)
import jax
import jax.numpy as jnp
from jax.experimental import pallas as pl


def kernel(oimage, aimage, olabel, alabel, oconf, aconf, prediction, cur_step):
    raise NotImplementedError("write your pallas kernel here")


# speedup vs baseline: 3.0548x; 3.0548x over previous
import jax
import jax.numpy as jnp
from jax import lax
from jax.experimental import pallas as pl
from jax.experimental.pallas import tpu as pltpu

BB = 4
IMG = 96
KP = 12
NBK = 8
LL = NBK ** 3
NCLS = 4
TOPK = 16
AGE_C = 1.0


def _stats_body(pred_ref, olab_ref, oconf_ref, aconf_ref,
                stats_ref, osum_ref, asum_ref):
    bi = pl.program_id(1)
    pred = pred_ref[0]
    lab = olab_ref[0]
    m = jnp.max(pred, axis=0)
    e = jnp.exp(pred - m[None])
    denom = jnp.sum(e, axis=0)
    inv = 1.0 / denom

    lanes = lax.broadcasted_iota(jnp.int32, (1, 1, 128), 2)
    acc = jnp.zeros((1, 1, 128), jnp.float32)
    for c in range(NCLS):
        p_c = e[c] * inv
        t_c = (lab == c).astype(jnp.float32)
        inter = jnp.sum(p_c * t_c)
        sump = jnp.sum(p_c)
        cnt = jnp.sum(t_c)
        acc = acc + jnp.where(lanes == c, inter, 0.0)
        acc = acc + jnp.where(lanes == NCLS + c, sump, 0.0)
        acc = acc + jnp.where(lanes == 2 * NCLS + c, cnt, 0.0)

    @pl.when(bi == 0)
    def _():
        stats_ref[...] = jnp.zeros_like(stats_ref)

    stats_ref[...] += acc

    zy = (lax.broadcasted_iota(jnp.int32, (NBK, IMG), 1) // KP
          == lax.broadcasted_iota(jnp.int32, (NBK, IMG), 0)).astype(jnp.float32)
    zz = (lax.broadcasted_iota(jnp.int32, (IMG, NBK), 0) // KP
          == lax.broadcasted_iota(jnp.int32, (IMG, NBK), 1)).astype(jnp.float32)

    hi = lax.Precision.HIGHEST
    ao = jnp.sum(oconf_ref[0], axis=0)
    aa = jnp.sum(aconf_ref[0], axis=0)
    po = jnp.dot(zy, jnp.dot(ao, zz, precision=hi,
                             preferred_element_type=jnp.float32),
                 precision=hi,
                 preferred_element_type=jnp.float32)
    pa = jnp.dot(zy, jnp.dot(aa, zz, precision=hi,
                             preferred_element_type=jnp.float32),
                 precision=hi,
                 preferred_element_type=jnp.float32)
    osum_ref[0, 0] = po
    asum_ref[0, 0] = pa


def _select_body(stats_ref, osum_ref, osumt_ref, asum_ref, asumt_ref,
                 osel_ref, asel_ref, tkv_ref):
    st = stats_ref[0]
    inter = st[:, 0:NCLS]
    sump = st[:, NCLS:2 * NCLS]
    cnt = st[:, 2 * NCLS:3 * NCLS]
    dice_terms = 2.0 * inter / (sump + cnt + 1e-5)
    dice = jnp.sum(dice_terms) / NCLS
    sl = 1.0 - dice
    sp_mask = sl < AGE_C
    sp_w = 1.0 - sl / (AGE_C + 1e-5)
    tk = jnp.minimum(TOPK, jnp.abs((TOPK * sp_w).astype(jnp.int32)))

    jj = lax.broadcasted_iota(jnp.int32, (LL, LL), 1)
    ii = lax.broadcasted_iota(jnp.int32, (LL, LL), 0)
    tsel = lax.broadcasted_iota(jnp.int32, (1, TOPK), 1)
    kk = lax.broadcasted_iota(jnp.int32, (LL, TOPK), 0)

    def ranks(row, col):
        lt = (row < col).astype(jnp.float32)
        tie = ((row == col) & (jj < ii)).astype(jnp.float32)
        return jnp.sum(lt + tie, axis=1, keepdims=True)

    o_rank = ranks(osum_ref[0], osumt_ref[0])
    a_rank = ranks(asum_ref[0], asumt_ref[0])

    o_tgt = jnp.where(sp_mask, (LL - 1) - tsel, tsel).astype(jnp.float32)
    a_tgt = jnp.where(sp_mask, tsel, (LL - 1) - tsel).astype(jnp.float32)

    osel = jnp.sum(jnp.where(o_rank == o_tgt, kk, 0), axis=0, keepdims=True)
    asel = jnp.sum(jnp.where(a_rank == a_tgt, kk, 0), axis=0, keepdims=True)

    osel_ref[0] = osel
    asel_ref[0] = asel
    tkv_ref[0] = jnp.zeros((1, TOPK), jnp.int32) + tk


def _swap_body(osel_ref, asel_ref, tk_ref,
               aimg, oimg, alab, olab, acf, ocf,
               oimg_o, olab_o, ocf_o,
               sem0, sem1, sem2):
    b = pl.program_id(0)
    t = pl.program_id(1)
    keep = t < tk_ref[b]
    s_o = osel_ref[b * TOPK + t]
    s_a = asel_ref[b * TOPK + t]
    xo = (s_o // (NBK * NBK)) * KP
    yo = ((s_o // NBK) % NBK) * KP
    zo = s_o % NBK
    xa = (s_a // (NBK * NBK)) * KP
    ya = ((s_a // NBK) % NBK) * KP
    za = s_a % NBK

    @pl.when(keep)
    def _():
        c0 = pltpu.make_async_copy(
            aimg.at[b, 0, pl.ds(xa, KP), pl.ds(ya, KP), za, :],
            oimg_o.at[b, 0, pl.ds(xo, KP), pl.ds(yo, KP), zo, :],
            sem0)
        c1 = pltpu.make_async_copy(
            alab.at[b, pl.ds(xa, KP), pl.ds(ya, KP), za, :],
            olab_o.at[b, pl.ds(xo, KP), pl.ds(yo, KP), zo, :],
            sem1)
        c2 = pltpu.make_async_copy(
            acf.at[b, pl.ds(xa, KP), pl.ds(ya, KP), za, :],
            ocf_o.at[b, pl.ds(xo, KP), pl.ds(yo, KP), zo, :],
            sem2)
        c0.start()
        c1.start()
        c2.start()
        c0.wait()
        c1.wait()
        c2.wait()


def kernel(oimage, aimage, olabel, alabel, oconf, aconf, prediction, cur_step):
    del cur_step
    f32 = jnp.float32
    i32 = jnp.int32

    stats, osum4, asum4 = pl.pallas_call(
        _stats_body,
        grid=(BB, NBK),
        in_specs=[
            pl.BlockSpec((1, NCLS, KP, IMG, IMG), lambda b, i: (b, 0, i, 0, 0)),
            pl.BlockSpec((1, KP, IMG, IMG), lambda b, i: (b, i, 0, 0)),
            pl.BlockSpec((1, KP, IMG, IMG), lambda b, i: (b, i, 0, 0)),
            pl.BlockSpec((1, KP, IMG, IMG), lambda b, i: (b, i, 0, 0)),
        ],
        out_specs=[
            pl.BlockSpec((1, 1, 128), lambda b, i: (b, 0, 0)),
            pl.BlockSpec((1, 1, NBK, NBK), lambda b, i: (b, i, 0, 0)),
            pl.BlockSpec((1, 1, NBK, NBK), lambda b, i: (b, i, 0, 0)),
        ],
        out_shape=[
            jax.ShapeDtypeStruct((BB, 1, 128), f32),
            jax.ShapeDtypeStruct((BB, NBK, NBK, NBK), f32),
            jax.ShapeDtypeStruct((BB, NBK, NBK, NBK), f32),
        ],
        compiler_params=pltpu.CompilerParams(
            dimension_semantics=("arbitrary", "arbitrary")),
    )(prediction, olabel, oconf, aconf)

    osum = osum4.reshape(BB, 1, LL)
    asum = asum4.reshape(BB, 1, LL)
    osumt = osum4.reshape(BB, LL, 1)
    asumt = asum4.reshape(BB, LL, 1)

    oselb, aselb, tkvb = pl.pallas_call(
        _select_body,
        grid=(BB,),
        in_specs=[
            pl.BlockSpec((1, 1, 128), lambda b: (b, 0, 0)),
            pl.BlockSpec((1, 1, LL), lambda b: (b, 0, 0)),
            pl.BlockSpec((1, LL, 1), lambda b: (b, 0, 0)),
            pl.BlockSpec((1, 1, LL), lambda b: (b, 0, 0)),
            pl.BlockSpec((1, LL, 1), lambda b: (b, 0, 0)),
        ],
        out_specs=[
            pl.BlockSpec((1, 1, TOPK), lambda b: (b, 0, 0)),
            pl.BlockSpec((1, 1, TOPK), lambda b: (b, 0, 0)),
            pl.BlockSpec((1, 1, TOPK), lambda b: (b, 0, 0)),
        ],
        out_shape=[
            jax.ShapeDtypeStruct((BB, 1, TOPK), i32),
            jax.ShapeDtypeStruct((BB, 1, TOPK), i32),
            jax.ShapeDtypeStruct((BB, 1, TOPK), i32),
        ],
        compiler_params=pltpu.CompilerParams(
            dimension_semantics=("arbitrary",)),
    )(stats, osum, osumt, asum, asumt)

    osel_flat = oselb.reshape(BB * TOPK)
    asel_flat = aselb.reshape(BB * TOPK)
    tkarr = tkvb[:, 0, 0]

    grid_spec = pltpu.PrefetchScalarGridSpec(
        num_scalar_prefetch=3,
        grid=(BB, TOPK),
        in_specs=[pl.BlockSpec(memory_space=pl.ANY)] * 6,
        out_specs=[pl.BlockSpec(memory_space=pl.ANY)] * 3,
        scratch_shapes=[pltpu.SemaphoreType.DMA] * 3,
    )

    sh6 = (BB, 1, IMG, IMG, NBK, KP)
    sh5 = (BB, IMG, IMG, NBK, KP)
    oimage_out, olabel_out, oconf_out = pl.pallas_call(
        _swap_body,
        grid_spec=grid_spec,
        out_shape=[
            jax.ShapeDtypeStruct(sh6, f32),
            jax.ShapeDtypeStruct(sh5, i32),
            jax.ShapeDtypeStruct(sh5, f32),
        ],
        input_output_aliases={4: 0, 6: 1, 8: 2},
        compiler_params=pltpu.CompilerParams(
            dimension_semantics=("arbitrary", "arbitrary")),
    )(osel_flat, asel_flat, tkarr,
      aimage.reshape(sh6), oimage.reshape(sh6),
      alabel.reshape(sh5), olabel.reshape(sh5),
      aconf.reshape(sh5), oconf.reshape(sh5))

    return (oimage_out.reshape(oimage.shape),
            olabel_out.reshape(olabel.shape),
            oconf_out.reshape(oconf.shape))


# speedup vs baseline: 3.0759x; 1.0069x over previous
import jax
import jax.numpy as jnp
from jax import lax
from jax.experimental import pallas as pl
from jax.experimental.pallas import tpu as pltpu
from jax.experimental.pallas import tpu_sc as plsc

BB = 4
IMG = 96
KP = 12
NBK = 8
LL = NBK ** 3
NCLS = 4
TOPK = 16
AGE_C = 1.0


def _stats_body(pred_ref, olab_ref, oconf_ref, aconf_ref,
                stats_ref, osum_ref, asum_ref):
    bi = pl.program_id(1)
    pred = pred_ref[0]
    lab = olab_ref[0]
    m = jnp.max(pred, axis=0)
    e = jnp.exp(pred - m[None])
    denom = jnp.sum(e, axis=0)
    inv = 1.0 / denom

    lanes = lax.broadcasted_iota(jnp.int32, (1, 1, 128), 2)
    acc = jnp.zeros((1, 1, 128), jnp.float32)
    for c in range(NCLS):
        p_c = e[c] * inv
        t_c = (lab == c).astype(jnp.float32)
        inter = jnp.sum(p_c * t_c)
        sump = jnp.sum(p_c)
        cnt = jnp.sum(t_c)
        acc = acc + jnp.where(lanes == c, inter, 0.0)
        acc = acc + jnp.where(lanes == NCLS + c, sump, 0.0)
        acc = acc + jnp.where(lanes == 2 * NCLS + c, cnt, 0.0)

    @pl.when(bi == 0)
    def _():
        stats_ref[...] = jnp.zeros_like(stats_ref)

    stats_ref[...] += acc

    zy = (lax.broadcasted_iota(jnp.int32, (NBK, IMG), 1) // KP
          == lax.broadcasted_iota(jnp.int32, (NBK, IMG), 0)).astype(jnp.float32)
    zz = (lax.broadcasted_iota(jnp.int32, (IMG, NBK), 0) // KP
          == lax.broadcasted_iota(jnp.int32, (IMG, NBK), 1)).astype(jnp.float32)

    hi = lax.Precision.HIGHEST
    ao = jnp.sum(oconf_ref[0], axis=0)
    aa = jnp.sum(aconf_ref[0], axis=0)
    po = jnp.dot(zy, jnp.dot(ao, zz, precision=hi,
                             preferred_element_type=jnp.float32),
                 precision=hi,
                 preferred_element_type=jnp.float32)
    pa = jnp.dot(zy, jnp.dot(aa, zz, precision=hi,
                             preferred_element_type=jnp.float32),
                 precision=hi,
                 preferred_element_type=jnp.float32)
    osum_ref[0, 0] = po
    asum_ref[0, 0] = pa


NCH = LL // 16
BIGF = 3e38


def _bfly(v, op):
    iota = lax.iota(jnp.int32, 16)
    for sh in (8, 4, 2, 1):
        v = op(v, v.at[iota ^ sh].get(mode="promise_in_bounds"))
    return v


def _sc_select_body(stats_hbm, osum_hbm, asum_hbm,
                    osel_hbm, asel_hbm, tkv_hbm,
                    st_v, keys_v, sel_v, tk_v):
    wid = lax.axis_index("s") * 2 + lax.axis_index("c")
    b = wid // 2
    is_o = (wid % 2) == 0
    iota = lax.iota(jnp.int32, 16)

    @pl.when(wid < 2 * BB)
    def _():
        pltpu.sync_copy(stats_hbm.at[b], st_v)
        inter = st_v[0]
        sump = st_v[1]
        cnt = st_v[2]
        terms = 2.0 * inter / (sump + cnt + 1e-5)
        dice = _bfly(terms, jnp.add) / NCLS
        sl = 1.0 - dice
        sp_w = 1.0 - sl / (AGE_C + 1e-5)
        tk16 = jnp.minimum(TOPK, jnp.abs((TOPK * sp_w).astype(jnp.int32)))
        spm = jnp.where(sl < AGE_C, 1.0, 0.0)
        iof = jnp.float32(1.0) * jnp.where(is_o, 1.0, 0.0)
        descind = spm * iof + (1.0 - spm) * (1.0 - iof)

        @pl.when(is_o)
        def _():
            pltpu.sync_copy(osum_hbm.at[b], keys_v)

        @pl.when(jnp.logical_not(is_o))
        def _():
            pltpu.sync_copy(asum_hbm.at[b], keys_v)

        sgn = 1.0 - 2.0 * descind
        descf = -sgn

        def pre(i, c):
            keys_v[pl.ds(i * 16, 16)] = keys_v[pl.ds(i * 16, 16)] * sgn
            return c
        lax.fori_loop(0, NCH, pre, 0)

        def step(t, sel):
            def mn(i, m):
                return jnp.minimum(m, keys_v[pl.ds(i * 16, 16)])
            m = lax.fori_loop(0, NCH, mn, jnp.full((16,), BIGF, jnp.float32))
            mval = _bfly(m, jnp.minimum)

            def bi(i, bst):
                ch = keys_v[pl.ds(i * 16, 16)]
                idxf = (iota + i * 16).astype(jnp.float32)
                jp = jnp.where(ch == mval, idxf * descf, -BIGF)
                return jnp.maximum(bst, jp)
            bst = lax.fori_loop(0, NCH, bi,
                                jnp.full((16,), -BIGF, jnp.float32))
            chosen = (_bfly(bst, jnp.maximum) * descf).astype(jnp.int32)
            sel = jnp.where(iota == t, chosen, sel)

            def mk(i, c):
                ch = keys_v[pl.ds(i * 16, 16)]
                keys_v[pl.ds(i * 16, 16)] = jnp.where(
                    iota + i * 16 == chosen, BIGF, ch)
                return c
            lax.fori_loop(0, NCH, mk, 0)
            return sel

        sel = lax.fori_loop(0, TOPK, step, jnp.zeros((16,), jnp.int32))
        sel_v[...] = sel

        @pl.when(is_o)
        def _():
            pltpu.sync_copy(sel_v, osel_hbm.at[b])
            tk_v[...] = tk16
            pltpu.sync_copy(tk_v, tkv_hbm.at[b])

        @pl.when(jnp.logical_not(is_o))
        def _():
            pltpu.sync_copy(sel_v, asel_hbm.at[b])


def _swap_body(osel_ref, asel_ref, tk_ref,
               aimg, oimg, alab, olab, acf, ocf,
               oimg_o, olab_o, ocf_o,
               sem0, sem1, sem2):
    b = pl.program_id(0)
    t = pl.program_id(1)
    keep = t < tk_ref[b]
    s_o = osel_ref[b * TOPK + t]
    s_a = asel_ref[b * TOPK + t]
    xo = (s_o // (NBK * NBK)) * KP
    yo = ((s_o // NBK) % NBK) * KP
    zo = s_o % NBK
    xa = (s_a // (NBK * NBK)) * KP
    ya = ((s_a // NBK) % NBK) * KP
    za = s_a % NBK

    @pl.when(keep)
    def _():
        c0 = pltpu.make_async_copy(
            aimg.at[b, 0, pl.ds(xa, KP), pl.ds(ya, KP), za, :],
            oimg_o.at[b, 0, pl.ds(xo, KP), pl.ds(yo, KP), zo, :],
            sem0)
        c1 = pltpu.make_async_copy(
            alab.at[b, pl.ds(xa, KP), pl.ds(ya, KP), za, :],
            olab_o.at[b, pl.ds(xo, KP), pl.ds(yo, KP), zo, :],
            sem1)
        c2 = pltpu.make_async_copy(
            acf.at[b, pl.ds(xa, KP), pl.ds(ya, KP), za, :],
            ocf_o.at[b, pl.ds(xo, KP), pl.ds(yo, KP), zo, :],
            sem2)
        c0.start()
        c1.start()
        c2.start()
        c0.wait()
        c1.wait()
        c2.wait()


def kernel(oimage, aimage, olabel, alabel, oconf, aconf, prediction, cur_step):
    del cur_step
    f32 = jnp.float32
    i32 = jnp.int32

    stats, osum4, asum4 = pl.pallas_call(
        _stats_body,
        grid=(BB, NBK),
        in_specs=[
            pl.BlockSpec((1, NCLS, KP, IMG, IMG), lambda b, i: (b, 0, i, 0, 0)),
            pl.BlockSpec((1, KP, IMG, IMG), lambda b, i: (b, i, 0, 0)),
            pl.BlockSpec((1, KP, IMG, IMG), lambda b, i: (b, i, 0, 0)),
            pl.BlockSpec((1, KP, IMG, IMG), lambda b, i: (b, i, 0, 0)),
        ],
        out_specs=[
            pl.BlockSpec((1, 1, 128), lambda b, i: (b, 0, 0)),
            pl.BlockSpec((1, 1, NBK, NBK), lambda b, i: (b, i, 0, 0)),
            pl.BlockSpec((1, 1, NBK, NBK), lambda b, i: (b, i, 0, 0)),
        ],
        out_shape=[
            jax.ShapeDtypeStruct((BB, 1, 128), f32),
            jax.ShapeDtypeStruct((BB, NBK, NBK, NBK), f32),
            jax.ShapeDtypeStruct((BB, NBK, NBK, NBK), f32),
        ],
        compiler_params=pltpu.CompilerParams(
            dimension_semantics=("arbitrary", "arbitrary")),
    )(prediction, olabel, oconf, aconf)

    osum2 = osum4.reshape(BB, LL)
    asum2 = asum4.reshape(BB, LL)
    st = stats[:, 0, :]
    z12 = jnp.zeros((BB, 12), f32)
    sc_stats = jnp.stack([
        jnp.concatenate([st[:, 0:NCLS], z12], axis=1),
        jnp.concatenate([st[:, NCLS:2 * NCLS], z12], axis=1),
        jnp.concatenate([st[:, 2 * NCLS:3 * NCLS], z12], axis=1),
    ], axis=1)

    oselb, aselb, tkvb = pl.kernel(
        _sc_select_body,
        out_type=[
            jax.ShapeDtypeStruct((BB, TOPK), i32),
            jax.ShapeDtypeStruct((BB, TOPK), i32),
            jax.ShapeDtypeStruct((BB, TOPK), i32),
        ],
        mesh=plsc.VectorSubcoreMesh(core_axis_name="c", subcore_axis_name="s"),
        scratch_types=[
            pltpu.VMEM((3, 16), f32),
            pltpu.VMEM((LL,), f32),
            pltpu.VMEM((16,), i32),
            pltpu.VMEM((16,), i32),
        ],
    )(sc_stats, osum2, asum2)

    osel_flat = oselb.reshape(BB * TOPK)
    asel_flat = aselb.reshape(BB * TOPK)
    tkarr = tkvb[:, 0]

    grid_spec = pltpu.PrefetchScalarGridSpec(
        num_scalar_prefetch=3,
        grid=(BB, TOPK),
        in_specs=[pl.BlockSpec(memory_space=pl.ANY)] * 6,
        out_specs=[pl.BlockSpec(memory_space=pl.ANY)] * 3,
        scratch_shapes=[pltpu.SemaphoreType.DMA] * 3,
    )

    sh6 = (BB, 1, IMG, IMG, NBK, KP)
    sh5 = (BB, IMG, IMG, NBK, KP)
    oimage_out, olabel_out, oconf_out = pl.pallas_call(
        _swap_body,
        grid_spec=grid_spec,
        out_shape=[
            jax.ShapeDtypeStruct(sh6, f32),
            jax.ShapeDtypeStruct(sh5, i32),
            jax.ShapeDtypeStruct(sh5, f32),
        ],
        input_output_aliases={4: 0, 6: 1, 8: 2},
        compiler_params=pltpu.CompilerParams(
            dimension_semantics=("arbitrary", "arbitrary")),
    )(osel_flat, asel_flat, tkarr,
      aimage.reshape(sh6), oimage.reshape(sh6),
      alabel.reshape(sh5), olabel.reshape(sh5),
      aconf.reshape(sh5), oconf.reshape(sh5))

    return (oimage_out.reshape(oimage.shape),
            olabel_out.reshape(olabel.shape),
            oconf_out.reshape(oconf.shape))


# speedup vs baseline: 3.1090x; 1.0108x over previous
import jax
import jax.numpy as jnp
from jax import lax
from jax.experimental import pallas as pl
from jax.experimental.pallas import tpu as pltpu
from jax.experimental.pallas import tpu_sc as plsc

BB = 4
IMG = 96
KP = 12
NBK = 8
LL = NBK ** 3
NCLS = 4
TOPK = 16
AGE_C = 1.0


def _stats_body(pred_ref, olab_ref, oconf_ref, aconf_ref,
                stats_ref, osum_ref, asum_ref):
    bi = pl.program_id(1)
    pred = pred_ref[0]
    lab = olab_ref[0]
    m = jnp.max(pred, axis=0)
    e = jnp.exp(pred - m[None])
    denom = jnp.sum(e, axis=0)
    inv = 1.0 / denom

    lanes = lax.broadcasted_iota(jnp.int32, (1, 1, 128), 2)
    acc = jnp.zeros((1, 1, 128), jnp.float32)
    for c in range(NCLS):
        p_c = e[c] * inv
        t_c = (lab == c).astype(jnp.float32)
        inter = jnp.sum(p_c * t_c)
        sump = jnp.sum(p_c)
        cnt = jnp.sum(t_c)
        acc = acc + jnp.where(lanes == c, inter, 0.0)
        acc = acc + jnp.where(lanes == NCLS + c, sump, 0.0)
        acc = acc + jnp.where(lanes == 2 * NCLS + c, cnt, 0.0)

    @pl.when(bi == 0)
    def _():
        stats_ref[...] = jnp.zeros_like(stats_ref)

    stats_ref[...] += acc

    zy = (lax.broadcasted_iota(jnp.int32, (NBK, IMG), 1) // KP
          == lax.broadcasted_iota(jnp.int32, (NBK, IMG), 0)).astype(jnp.float32)
    zz = (lax.broadcasted_iota(jnp.int32, (IMG, NBK), 0) // KP
          == lax.broadcasted_iota(jnp.int32, (IMG, NBK), 1)).astype(jnp.float32)

    hi = lax.Precision.HIGHEST
    ao = jnp.sum(oconf_ref[0], axis=0)
    aa = jnp.sum(aconf_ref[0], axis=0)
    po = jnp.dot(zy, jnp.dot(ao, zz, precision=hi,
                             preferred_element_type=jnp.float32),
                 precision=hi,
                 preferred_element_type=jnp.float32)
    pa = jnp.dot(zy, jnp.dot(aa, zz, precision=hi,
                             preferred_element_type=jnp.float32),
                 precision=hi,
                 preferred_element_type=jnp.float32)
    osum_ref[0, 0] = po
    asum_ref[0, 0] = pa


NCH = LL // 16
BIGF = 3e38


def _bfly(v, op):
    iota = lax.iota(jnp.int32, 16)
    for sh in (8, 4, 2, 1):
        v = op(v, v.at[iota ^ sh].get(mode="promise_in_bounds"))
    return v


def _sc_select_body(stats_hbm, osum_hbm, asum_hbm,
                    osel_hbm, asel_hbm, tkv_hbm,
                    st_v, keys_v, sel_v, tk_v):
    wid = lax.axis_index("s") * 2 + lax.axis_index("c")
    b = wid // 2
    is_o = (wid % 2) == 0
    iota = lax.iota(jnp.int32, 16)

    @pl.when(wid < 2 * BB)
    def _():
        pltpu.sync_copy(stats_hbm.at[b], st_v)
        inter = st_v[0]
        sump = st_v[1]
        cnt = st_v[2]
        terms = 2.0 * inter / (sump + cnt + 1e-5)
        dice = _bfly(terms, jnp.add) / NCLS
        sl = 1.0 - dice
        sp_w = 1.0 - sl / (AGE_C + 1e-5)
        tk16 = jnp.minimum(TOPK, jnp.abs((TOPK * sp_w).astype(jnp.int32)))
        spm = jnp.where(sl < AGE_C, 1.0, 0.0)
        iof = jnp.float32(1.0) * jnp.where(is_o, 1.0, 0.0)
        descind = spm * iof + (1.0 - spm) * (1.0 - iof)

        @pl.when(is_o)
        def _():
            pltpu.sync_copy(osum_hbm.at[b], keys_v)

        @pl.when(jnp.logical_not(is_o))
        def _():
            pltpu.sync_copy(asum_hbm.at[b], keys_v)

        sgn = 1.0 - 2.0 * descind
        descf = -sgn

        def pre(i, c):
            keys_v[pl.ds(i * 16, 16)] = keys_v[pl.ds(i * 16, 16)] * sgn
            return c
        lax.fori_loop(0, NCH, pre, 0)

        def step(t, sel):
            def mn(i, m):
                return jnp.minimum(m, keys_v[pl.ds(i * 16, 16)])
            m = lax.fori_loop(0, NCH, mn, jnp.full((16,), BIGF, jnp.float32))
            mval = _bfly(m, jnp.minimum)

            def bi(i, bst):
                ch = keys_v[pl.ds(i * 16, 16)]
                idxf = (iota + i * 16).astype(jnp.float32)
                jp = jnp.where(ch == mval, idxf * descf, -BIGF)
                return jnp.maximum(bst, jp)
            bst = lax.fori_loop(0, NCH, bi,
                                jnp.full((16,), -BIGF, jnp.float32))
            chosen = (_bfly(bst, jnp.maximum) * descf).astype(jnp.int32)
            sel = jnp.where(iota == t, chosen, sel)

            def mk(i, c):
                ch = keys_v[pl.ds(i * 16, 16)]
                keys_v[pl.ds(i * 16, 16)] = jnp.where(
                    iota + i * 16 == chosen, BIGF, ch)
                return c
            lax.fori_loop(0, NCH, mk, 0)
            return sel

        sel = lax.fori_loop(0, TOPK, step, jnp.zeros((16,), jnp.int32))
        sel_v[...] = sel

        @pl.when(is_o)
        def _():
            pltpu.sync_copy(sel_v, osel_hbm.at[b])
            tk_v[...] = tk16
            pltpu.sync_copy(tk_v, tkv_hbm.at[b])

        @pl.when(jnp.logical_not(is_o))
        def _():
            pltpu.sync_copy(sel_v, asel_hbm.at[b])


def _swap_body(osel_ref, asel_ref, tk_ref,
               aimg, oimg, alab, olab, acf, ocf,
               oimg_o, olab_o, ocf_o,
               sem0, sem1, sem2):
    b = pl.program_id(0)
    tk = tk_ref[b]
    pend = []
    for t in range(TOPK):
        keep = t < tk
        s_o = osel_ref[b * TOPK + t]
        s_a = asel_ref[b * TOPK + t]
        xo = (s_o // (NBK * NBK)) * KP
        yo = ((s_o // NBK) % NBK) * KP
        zo = s_o % NBK
        xa = (s_a // (NBK * NBK)) * KP
        ya = ((s_a // NBK) % NBK) * KP
        za = s_a % NBK
        c0 = pltpu.make_async_copy(
            aimg.at[b, 0, pl.ds(xa, KP), pl.ds(ya, KP), za, :],
            oimg_o.at[b, 0, pl.ds(xo, KP), pl.ds(yo, KP), zo, :],
            sem0)
        c1 = pltpu.make_async_copy(
            alab.at[b, pl.ds(xa, KP), pl.ds(ya, KP), za, :],
            olab_o.at[b, pl.ds(xo, KP), pl.ds(yo, KP), zo, :],
            sem1)
        c2 = pltpu.make_async_copy(
            acf.at[b, pl.ds(xa, KP), pl.ds(ya, KP), za, :],
            ocf_o.at[b, pl.ds(xo, KP), pl.ds(yo, KP), zo, :],
            sem2)

        @pl.when(keep)
        def _(c0=c0, c1=c1, c2=c2):
            c0.start()
            c1.start()
            c2.start()
        pend.append((keep, c0, c1, c2))

    for keep, c0, c1, c2 in pend:
        @pl.when(keep)
        def _(c0=c0, c1=c1, c2=c2):
            c0.wait()
            c1.wait()
            c2.wait()


def kernel(oimage, aimage, olabel, alabel, oconf, aconf, prediction, cur_step):
    del cur_step
    f32 = jnp.float32
    i32 = jnp.int32

    stats, osum4, asum4 = pl.pallas_call(
        _stats_body,
        grid=(BB, NBK),
        in_specs=[
            pl.BlockSpec((1, NCLS, KP, IMG, IMG), lambda b, i: (b, 0, i, 0, 0)),
            pl.BlockSpec((1, KP, IMG, IMG), lambda b, i: (b, i, 0, 0)),
            pl.BlockSpec((1, KP, IMG, IMG), lambda b, i: (b, i, 0, 0)),
            pl.BlockSpec((1, KP, IMG, IMG), lambda b, i: (b, i, 0, 0)),
        ],
        out_specs=[
            pl.BlockSpec((1, 1, 128), lambda b, i: (b, 0, 0)),
            pl.BlockSpec((1, 1, NBK, NBK), lambda b, i: (b, i, 0, 0)),
            pl.BlockSpec((1, 1, NBK, NBK), lambda b, i: (b, i, 0, 0)),
        ],
        out_shape=[
            jax.ShapeDtypeStruct((BB, 1, 128), f32),
            jax.ShapeDtypeStruct((BB, NBK, NBK, NBK), f32),
            jax.ShapeDtypeStruct((BB, NBK, NBK, NBK), f32),
        ],
        compiler_params=pltpu.CompilerParams(
            dimension_semantics=("arbitrary", "arbitrary")),
    )(prediction, olabel, oconf, aconf)

    osum2 = osum4.reshape(BB, LL)
    asum2 = asum4.reshape(BB, LL)
    st = stats[:, 0, :]
    z12 = jnp.zeros((BB, 12), f32)
    sc_stats = jnp.stack([
        jnp.concatenate([st[:, 0:NCLS], z12], axis=1),
        jnp.concatenate([st[:, NCLS:2 * NCLS], z12], axis=1),
        jnp.concatenate([st[:, 2 * NCLS:3 * NCLS], z12], axis=1),
    ], axis=1)

    oselb, aselb, tkvb = pl.kernel(
        _sc_select_body,
        out_type=[
            jax.ShapeDtypeStruct((BB, TOPK), i32),
            jax.ShapeDtypeStruct((BB, TOPK), i32),
            jax.ShapeDtypeStruct((BB, TOPK), i32),
        ],
        mesh=plsc.VectorSubcoreMesh(core_axis_name="c", subcore_axis_name="s"),
        scratch_types=[
            pltpu.VMEM((3, 16), f32),
            pltpu.VMEM((LL,), f32),
            pltpu.VMEM((16,), i32),
            pltpu.VMEM((16,), i32),
        ],
    )(sc_stats, osum2, asum2)

    osel_flat = oselb.reshape(BB * TOPK)
    asel_flat = aselb.reshape(BB * TOPK)
    tkarr = tkvb[:, 0]

    grid_spec = pltpu.PrefetchScalarGridSpec(
        num_scalar_prefetch=3,
        grid=(BB,),
        in_specs=[pl.BlockSpec(memory_space=pl.ANY)] * 6,
        out_specs=[pl.BlockSpec(memory_space=pl.ANY)] * 3,
        scratch_shapes=[pltpu.SemaphoreType.DMA] * 3,
    )

    sh6 = (BB, 1, IMG, IMG, NBK, KP)
    sh5 = (BB, IMG, IMG, NBK, KP)
    oimage_out, olabel_out, oconf_out = pl.pallas_call(
        _swap_body,
        grid_spec=grid_spec,
        out_shape=[
            jax.ShapeDtypeStruct(sh6, f32),
            jax.ShapeDtypeStruct(sh5, i32),
            jax.ShapeDtypeStruct(sh5, f32),
        ],
        input_output_aliases={4: 0, 6: 1, 8: 2},
        compiler_params=pltpu.CompilerParams(
            dimension_semantics=("arbitrary",)),
    )(osel_flat, asel_flat, tkarr,
      aimage.reshape(sh6), oimage.reshape(sh6),
      alabel.reshape(sh5), olabel.reshape(sh5),
      aconf.reshape(sh5), oconf.reshape(sh5))

    return (oimage_out.reshape(oimage.shape),
            olabel_out.reshape(olabel.shape),
            oconf_out.reshape(oconf.shape))
